# Initial kernel scaffold; baseline (speedup 1.0000x reference)
import jax
import jax.numpy as jnp
from jax.experimental import pallas as pl


def kernel(v_t, p_t, stored_tokens, qp_w1, qp_b1, qp_w2, qp_b2, pd_w1, pd_b1, pd_w2, pd_b2, pd_w3, pd_b3, dd_w1, dd_b1, dd_w2, dd_b2, k):
    raise NotImplementedError("write your pallas kernel here")


# speedup vs baseline: 1.7083x; 1.7083x over previous
import functools

import jax
import jax.numpy as jnp
from jax import lax
from jax.experimental import pallas as pl
from jax.experimental.pallas import tpu as pltpu
from jax.experimental.pallas import tpu_sc as plsc

EMB = 64
POS = 3
TOK = 2 * EMB
QN = 1024
KMEM = 100000
TOPK = 16
KBLK = 1024
NKB = 98
KPAD = NKB * KBLK
QT = 256
NQT = QN // QT
DQT = 2048
BIG = 3.0e38
IBIG = 2**30


def _proj_body(qin_ref, w1_ref, b1_ref, w2_ref, b2_ref, out_ref):
    mm = lambda a, b: lax.dot_general(a, b, (((1,), (0,)), ((), ())),
                                      preferred_element_type=jnp.float32)
    h = jnp.maximum(mm(qin_ref[...], w1_ref[...]) + b1_ref[...], 0.0)
    out_ref[...] = mm(h, w2_ref[...]) + b2_ref[...]


def _run_proj(qin, w1, b1, w2, b2):
    return pl.pallas_call(
        _proj_body,
        out_shape=jax.ShapeDtypeStruct((QN, TOK), jnp.float32),
    )(qin, w1, b1, w2, b2)


def _topk_body(qp_ref, st_ref, ksq_ref, vals_ref, idx_ref, bv_ref, bi_ref):
    j = pl.program_id(1)
    qp = qp_ref[...]
    st = st_ref[...]
    qk = lax.dot_general(qp, st, (((1,), (1,)), ((), ())),
                         preferred_element_type=jnp.float32)
    qsq = jnp.sum(qp * qp, axis=1, keepdims=True)
    ksq = ksq_ref[...].reshape(1, KBLK)
    d2 = (qsq - 2.0 * qk) + ksq
    gidx = j * KBLK + lax.broadcasted_iota(jnp.int32, (QT, KBLK), 1)
    d2 = jnp.where(gidx < KMEM, d2, BIG)

    first = j == 0
    bv = jnp.where(first, BIG, bv_ref[...])
    bi = jnp.where(first, IBIG, bi_ref[...])

    cv = jnp.concatenate([bv, d2], axis=1)
    ci = jnp.concatenate([bi, gidx], axis=1)
    lane = lax.broadcasted_iota(jnp.int32, (QT, 128), 1)
    nv = jnp.full((QT, 128), BIG, jnp.float32)
    ni = jnp.full((QT, 128), IBIG, jnp.int32)
    for t in range(TOPK):
        m = jnp.min(cv, axis=1, keepdims=True)
        sel = jnp.min(jnp.where(cv == m, ci, IBIG), axis=1, keepdims=True)
        cv = jnp.where(ci == sel, BIG, cv)
        nv = jnp.where(lane == t, m, nv)
        ni = jnp.where(lane == t, sel, ni)
    bv_ref[...] = nv
    bi_ref[...] = ni

    @pl.when(j == NKB - 1)
    def _():
        vals_ref[...] = nv
        idx_ref[...] = ni


def _run_topk(qproj, st_p, ksq_p):
    return pl.pallas_call(
        _topk_body,
        grid=(NQT, NKB),
        in_specs=[
            pl.BlockSpec((QT, TOK), lambda i, j: (i, 0)),
            pl.BlockSpec((KBLK, TOK), lambda i, j: (j, 0)),
            pl.BlockSpec((1, 1, KBLK), lambda i, j: (j, 0, 0)),
        ],
        out_specs=[
            pl.BlockSpec((QT, 128), lambda i, j: (i, 0)),
            pl.BlockSpec((QT, 128), lambda i, j: (i, 0)),
        ],
        out_shape=[
            jax.ShapeDtypeStruct((QN, 128), jnp.float32),
            jax.ShapeDtypeStruct((QN, 128), jnp.int32),
        ],
        scratch_shapes=[
            pltpu.VMEM((QT, 128), jnp.float32),
            pltpu.VMEM((QT, 128), jnp.int32),
        ],
        compiler_params=pltpu.CompilerParams(
            dimension_semantics=("arbitrary", "arbitrary")),
    )(qproj, st_p, ksq_p)


_NW = 32
_BPW = QN * TOPK // _NW
_CHUNK = 128


def _make_gather():
    mesh = plsc.VectorSubcoreMesh(core_axis_name="c", subcore_axis_name="s")

    @functools.partial(
        pl.kernel, mesh=mesh,
        out_type=jax.ShapeDtypeStruct((QN * TOPK, TOK), jnp.float32),
        scratch_types=[
            pltpu.VMEM((_BPW,), jnp.int32),
            pltpu.VMEM((_BPW, TOK), jnp.float32),
            pltpu.SemaphoreType.DMA,
        ],
    )
    def gather_kernel(table_hbm, idx_hbm, out_hbm, idx_v, rows_v, sem):
        wid = lax.axis_index("s") * 2 + lax.axis_index("c")
        base = wid * _BPW
        pltpu.sync_copy(idx_hbm.at[pl.ds(base, _BPW)], idx_v)
        cps = []
        for c in range(_BPW // _CHUNK):
            cps.append(pltpu.async_copy(
                table_hbm.at[idx_v.at[pl.ds(c * _CHUNK, _CHUNK)]],
                rows_v.at[pl.ds(c * _CHUNK, _CHUNK)], sem))
        for cp in cps:
            cp.wait()
        pltpu.sync_copy(rows_v, out_hbm.at[pl.ds(base, _BPW)])

    return gather_kernel


def _dec_body(vs_ref, pw1, pb1, pw2, pb2, pw3, pb3, dw1, db1, dw2, db2,
              p_ref, d_ref):
    mm = lambda a, b: lax.dot_general(a, b, (((1,), (0,)), ((), ())),
                                      preferred_element_type=jnp.float32)
    vs = vs_ref[...]
    hp = jnp.maximum(mm(vs, pw1[...]) + pb1[...], 0.0)
    hp = jnp.maximum(mm(hp, pw2[...]) + pb2[...], 0.0)
    p_ref[...] = mm(hp, pw3[...]) + pb3[...]
    hd = jnp.maximum(mm(vs, dw1[...]) + db1[...], 0.0)
    d_ref[...] = mm(hd, dw2[...]) + db2[...]


def _run_dec(vs, pw1, pb1, pw2, pb2, pw3, pb3, dw1, db1, dw2, db2):
    n = QN * TOPK
    wspec = pl.BlockSpec((TOK, TOK), lambda i: (0, 0))
    bspec = pl.BlockSpec((1, TOK), lambda i: (0, 0))
    return pl.pallas_call(
        _dec_body,
        grid=(n // DQT,),
        in_specs=[pl.BlockSpec((DQT, TOK), lambda i: (i, 0)),
                  wspec, bspec, wspec, bspec, wspec, bspec,
                  wspec, bspec, wspec, bspec],
        out_specs=[pl.BlockSpec((DQT, TOK), lambda i: (i, 0)),
                   pl.BlockSpec((DQT, TOK), lambda i: (i, 0))],
        out_shape=[jax.ShapeDtypeStruct((n, TOK), jnp.float32),
                   jax.ShapeDtypeStruct((n, TOK), jnp.float32)],
    )(vs, pw1, pb1, pw2, pb2, pw3, pb3, dw1, db1, dw2, db2)


def _padw(w):
    return jnp.pad(w, ((0, TOK - w.shape[0]), (0, TOK - w.shape[1])))


def _padb(b):
    return jnp.pad(b, (0, TOK - b.shape[0]))[None, :]


def kernel(v_t, p_t, stored_tokens, qp_w1, qp_b1, qp_w2, qp_b2,
           pd_w1, pd_b1, pd_w2, pd_b2, pd_w3, pd_b3,
           dd_w1, dd_b1, dd_w2, dd_b2, k):
    qin = jnp.concatenate([v_t, p_t], axis=1)
    qin = jnp.pad(qin, ((0, 0), (0, TOK - EMB - POS)))
    w1 = jnp.pad(qp_w1, ((0, TOK - EMB - POS), (0, 0)))
    qproj = _run_proj(qin, w1, qp_b1[None, :], qp_w2, qp_b2[None, :])

    ksq = jnp.sum(stored_tokens * stored_tokens, axis=1)
    ksq_p = jnp.pad(ksq, (0, KPAD - KMEM)).reshape(NKB, 1, KBLK)
    st_p = jnp.pad(stored_tokens, ((0, KPAD - KMEM), (0, 0)))

    vals128, idx128 = _run_topk(qproj, st_p, ksq_p)
    vals = vals128[:, :TOPK]
    idx = idx128[:, :TOPK]

    retrieved = _make_gather()(stored_tokens, idx.reshape(-1))
    retrieved = retrieved.reshape(QN, TOPK, TOK)

    vs = retrieved[..., EMB:].reshape(QN * TOPK, EMB)
    vs = jnp.pad(vs, ((0, 0), (0, TOK - EMB)))
    p128, d128 = _run_dec(
        vs,
        _padw(pd_w1), _padb(pd_b1), _padw(pd_w2), _padb(pd_b2),
        _padw(pd_w3), _padb(pd_b3),
        _padw(dd_w1), _padb(dd_b1), _padw(dd_w2), _padb(dd_b2))
    p_hat = p128[:, :POS].reshape(QN, TOPK, POS)
    d_hat = d128[:, :EMB].reshape(QN, TOPK, EMB)
    return (idx, vals, retrieved, p_hat, d_hat)


# speedup vs baseline: 3.7682x; 2.2059x over previous
import functools

import jax
import jax.numpy as jnp
from jax import lax
from jax.experimental import pallas as pl
from jax.experimental.pallas import tpu as pltpu
from jax.experimental.pallas import tpu_sc as plsc

EMB = 64
POS = 3
TOK = 2 * EMB
QN = 1024
KMEM = 100000
TOPK = 16
KBLK = 1024
NKB = 98
KPAD = NKB * KBLK
QT = 256
NQT = QN // QT
DQT = 2048
BIG = 3.0e38
IBIG = 2**30


def _proj_body(qin_ref, w1_ref, b1_ref, w2_ref, b2_ref, out_ref):
    mm = lambda a, b: lax.dot_general(a, b, (((1,), (0,)), ((), ())),
                                      preferred_element_type=jnp.float32)
    h = jnp.maximum(mm(qin_ref[...], w1_ref[...]) + b1_ref[...], 0.0)
    out_ref[...] = mm(h, w2_ref[...]) + b2_ref[...]


def _run_proj(qin, w1, b1, w2, b2):
    return pl.pallas_call(
        _proj_body,
        out_shape=jax.ShapeDtypeStruct((QN, TOK), jnp.float32),
    )(qin, w1, b1, w2, b2)


def _topk_body(qp_ref, st_ref, ksq_ref, vals_ref, idx_ref,
               bv_ref, bi_ref, d2_ref):
    j = pl.program_id(1)
    qp = qp_ref[...]
    st = st_ref[...]
    qk = lax.dot_general(qp, st, (((1,), (1,)), ((), ())),
                         preferred_element_type=jnp.float32)
    qsq = jnp.sum(qp * qp, axis=1, keepdims=True)
    ksq = ksq_ref[...].reshape(1, KBLK)
    d2 = (qsq - 2.0 * qk) + ksq
    gidx = j * KBLK + lax.broadcasted_iota(jnp.int32, (QT, KBLK), 1)
    d2 = jnp.where(gidx < KMEM, d2, BIG)

    first = j == 0
    bv_ref[...] = jnp.where(first, BIG, bv_ref[...])
    bi_ref[...] = jnp.where(first, IBIG, bi_ref[...])

    lane = lax.broadcasted_iota(jnp.int32, (QT, 128), 1)
    lane16 = lane < TOPK

    tau = jnp.max(jnp.where(lane16, bv_ref[...], -BIG), axis=1, keepdims=True)
    cnt = jnp.sum((d2 < tau).astype(jnp.int32), axis=1, keepdims=True)
    npass = jnp.minimum(jnp.max(cnt), TOPK)
    d2_ref[...] = d2

    def body(t, carry):
        d2v = d2_ref[...]
        m = jnp.min(d2v, axis=1, keepdims=True)
        sel = jnp.min(jnp.where(d2v == m, gidx, IBIG), axis=1, keepdims=True)
        d2_ref[...] = jnp.where(gidx == sel, BIG, d2v)
        bvv = bv_ref[...]
        biv = bi_ref[...]
        bvm = jnp.where(lane16, bvv, -BIG)
        worst = jnp.max(bvm, axis=1, keepdims=True)
        slot = jnp.min(jnp.where(bvm == worst, lane, 128), axis=1,
                       keepdims=True)
        wmask = (lane == slot) & (m < worst)
        bv_ref[...] = jnp.where(wmask, m, bvv)
        bi_ref[...] = jnp.where(wmask, sel, biv)
        return carry

    lax.fori_loop(0, npass, body, 0)

    @pl.when(j == NKB - 1)
    def _():
        cv = jnp.where(lane16, bv_ref[...], BIG)
        ci = bi_ref[...]
        nv = jnp.full((QT, 128), BIG, jnp.float32)
        ni = jnp.full((QT, 128), IBIG, jnp.int32)
        for t in range(TOPK):
            m = jnp.min(cv, axis=1, keepdims=True)
            sel = jnp.min(jnp.where(cv == m, ci, IBIG), axis=1, keepdims=True)
            cv = jnp.where(ci == sel, BIG, cv)
            nv = jnp.where(lane == t, m, nv)
            ni = jnp.where(lane == t, sel, ni)
        vals_ref[...] = nv
        idx_ref[...] = ni


def _run_topk(qproj, st_p, ksq_p):
    return pl.pallas_call(
        _topk_body,
        grid=(NQT, NKB),
        in_specs=[
            pl.BlockSpec((QT, TOK), lambda i, j: (i, 0)),
            pl.BlockSpec((KBLK, TOK), lambda i, j: (j, 0)),
            pl.BlockSpec((1, 1, KBLK), lambda i, j: (j, 0, 0)),
        ],
        out_specs=[
            pl.BlockSpec((QT, 128), lambda i, j: (i, 0)),
            pl.BlockSpec((QT, 128), lambda i, j: (i, 0)),
        ],
        out_shape=[
            jax.ShapeDtypeStruct((QN, 128), jnp.float32),
            jax.ShapeDtypeStruct((QN, 128), jnp.int32),
        ],
        scratch_shapes=[
            pltpu.VMEM((QT, 128), jnp.float32),
            pltpu.VMEM((QT, 128), jnp.int32),
            pltpu.VMEM((QT, KBLK), jnp.float32),
        ],
        compiler_params=pltpu.CompilerParams(
            dimension_semantics=("arbitrary", "arbitrary")),
    )(qproj, st_p, ksq_p)


_NW = 32
_BPW = QN * TOPK // _NW
_CHUNK = 128


def _make_gather():
    mesh = plsc.VectorSubcoreMesh(core_axis_name="c", subcore_axis_name="s")

    @functools.partial(
        pl.kernel, mesh=mesh,
        out_type=jax.ShapeDtypeStruct((QN * TOPK, TOK), jnp.float32),
        scratch_types=[
            pltpu.VMEM((_BPW,), jnp.int32),
            pltpu.VMEM((_BPW, TOK), jnp.float32),
            pltpu.SemaphoreType.DMA,
        ],
    )
    def gather_kernel(table_hbm, idx_hbm, out_hbm, idx_v, rows_v, sem):
        wid = lax.axis_index("s") * 2 + lax.axis_index("c")
        base = wid * _BPW
        pltpu.sync_copy(idx_hbm.at[pl.ds(base, _BPW)], idx_v)
        cps = []
        for c in range(_BPW // _CHUNK):
            cps.append(pltpu.async_copy(
                table_hbm.at[idx_v.at[pl.ds(c * _CHUNK, _CHUNK)]],
                rows_v.at[pl.ds(c * _CHUNK, _CHUNK)], sem))
        for cp in cps:
            cp.wait()
        pltpu.sync_copy(rows_v, out_hbm.at[pl.ds(base, _BPW)])

    return gather_kernel


def _dec_body(vs_ref, pw1, pb1, pw2, pb2, pw3, pb3, dw1, db1, dw2, db2,
              p_ref, d_ref):
    mm = lambda a, b: lax.dot_general(a, b, (((1,), (0,)), ((), ())),
                                      preferred_element_type=jnp.float32)
    vs = vs_ref[...]
    hp = jnp.maximum(mm(vs, pw1[...]) + pb1[...], 0.0)
    hp = jnp.maximum(mm(hp, pw2[...]) + pb2[...], 0.0)
    p_ref[...] = mm(hp, pw3[...]) + pb3[...]
    hd = jnp.maximum(mm(vs, dw1[...]) + db1[...], 0.0)
    d_ref[...] = mm(hd, dw2[...]) + db2[...]


def _run_dec(vs, pw1, pb1, pw2, pb2, pw3, pb3, dw1, db1, dw2, db2):
    n = QN * TOPK
    wspec = pl.BlockSpec((TOK, TOK), lambda i: (0, 0))
    bspec = pl.BlockSpec((1, TOK), lambda i: (0, 0))
    return pl.pallas_call(
        _dec_body,
        grid=(n // DQT,),
        in_specs=[pl.BlockSpec((DQT, TOK), lambda i: (i, 0)),
                  wspec, bspec, wspec, bspec, wspec, bspec,
                  wspec, bspec, wspec, bspec],
        out_specs=[pl.BlockSpec((DQT, TOK), lambda i: (i, 0)),
                   pl.BlockSpec((DQT, TOK), lambda i: (i, 0))],
        out_shape=[jax.ShapeDtypeStruct((n, TOK), jnp.float32),
                   jax.ShapeDtypeStruct((n, TOK), jnp.float32)],
    )(vs, pw1, pb1, pw2, pb2, pw3, pb3, dw1, db1, dw2, db2)


def _padw(w):
    return jnp.pad(w, ((0, TOK - w.shape[0]), (0, TOK - w.shape[1])))


def _padb(b):
    return jnp.pad(b, (0, TOK - b.shape[0]))[None, :]


def kernel(v_t, p_t, stored_tokens, qp_w1, qp_b1, qp_w2, qp_b2,
           pd_w1, pd_b1, pd_w2, pd_b2, pd_w3, pd_b3,
           dd_w1, dd_b1, dd_w2, dd_b2, k):
    qin = jnp.concatenate([v_t, p_t], axis=1)
    qin = jnp.pad(qin, ((0, 0), (0, TOK - EMB - POS)))
    w1 = jnp.pad(qp_w1, ((0, TOK - EMB - POS), (0, 0)))
    qproj = _run_proj(qin, w1, qp_b1[None, :], qp_w2, qp_b2[None, :])

    ksq = jnp.sum(stored_tokens * stored_tokens, axis=1)
    ksq_p = jnp.pad(ksq, (0, KPAD - KMEM)).reshape(NKB, 1, KBLK)
    st_p = jnp.pad(stored_tokens, ((0, KPAD - KMEM), (0, 0)))

    vals128, idx128 = _run_topk(qproj, st_p, ksq_p)
    vals = vals128[:, :TOPK]
    idx = idx128[:, :TOPK]

    retrieved = _make_gather()(stored_tokens, idx.reshape(-1))
    retrieved = retrieved.reshape(QN, TOPK, TOK)

    vs = retrieved[..., EMB:].reshape(QN * TOPK, EMB)
    vs = jnp.pad(vs, ((0, 0), (0, TOK - EMB)))
    p128, d128 = _run_dec(
        vs,
        _padw(pd_w1), _padb(pd_b1), _padw(pd_w2), _padb(pd_b2),
        _padw(pd_w3), _padb(pd_b3),
        _padw(dd_w1), _padb(dd_b1), _padw(dd_w2), _padb(dd_b2))
    p_hat = p128[:, :POS].reshape(QN, TOPK, POS)
    d_hat = d128[:, :EMB].reshape(QN, TOPK, EMB)
    return (idx, vals, retrieved, p_hat, d_hat)
